# baseline (speedup 1.0000x reference)
import functools

import jax
import jax.numpy as jnp
from jax import lax
from jax.experimental import pallas as pl
from jax.experimental.pallas import tpu as pltpu
from jax.experimental.pallas import tpu_sc as plsc

_LANES = 16
_RC = 128


def _sc_gather(idx_flat, offs, tab2, D, nc, ns):
    BF = idx_flat.shape[0]
    W = tab2.shape[-1]
    nw = nc * ns
    epw = BF // nw
    nvec = epw // _LANES
    npair = epw // (2 * _RC)
    mesh = plsc.VectorSubcoreMesh(core_axis_name="c", subcore_axis_name="s")

    @functools.partial(
        pl.kernel,
        mesh=mesh,
        out_type=jax.ShapeDtypeStruct((BF, D), jnp.float32),
        scratch_types=[
            pltpu.VMEM((epw,), jnp.int32),
            pltpu.VMEM((epw,), jnp.int32),
            pltpu.VMEM((_RC, 8 * D), jnp.float32),
            pltpu.VMEM((_RC, 8 * D), jnp.float32),
            pltpu.VMEM((_RC, D), jnp.float32),
            pltpu.VMEM((_RC, D), jnp.float32),
            pltpu.SemaphoreType.DMA,
            pltpu.SemaphoreType.DMA,
            pltpu.SemaphoreType.DMA,
            pltpu.SemaphoreType.DMA,
        ],
        compiler_params=pltpu.CompilerParams(needs_layout_passes=False),
    )
    def gather_kernel(idx_hbm, offs_hbm, tab_hbm, x_hbm,
                      slab_v, sel_v, raw0, raw1, out0, out1,
                      g0, g1, w0, w1):
        wid = lax.axis_index("s") * nc + lax.axis_index("c")
        base = wid * epw
        pltpu.sync_copy(idx_hbm.at[pl.ds(base, epw)], slab_v)
        pltpu.sync_copy(offs_hbm, sel_v)

        lane = lax.iota(jnp.int32, _LANES)

        def prep(k, carry):
            o = k * _LANES
            g = slab_v[pl.ds(o, _LANES)] + sel_v[pl.ds(o, _LANES)]
            slab_v[pl.ds(o, _LANES)] = lax.shift_right_logical(g, 3)
            sel_v[pl.ds(o, _LANES)] = (g & 7) * D
            return carry

        lax.fori_loop(0, nvec, prep, 0)

        raws = (raw0, raw1)
        outs = (out0, out1)
        gsem = (g0, g1)
        wsem = (w0, w1)
        fulld = [jnp.full((_LANES,), d, jnp.int32) for d in range(D)]

        def select(c, raw, out):
            def group(g, carry):
                ivec = g * _LANES + lane
                remv = sel_v[pl.ds(c * _RC + g * _LANES, _LANES)]
                for d in range(D):
                    col = plsc.load_gather(raw, [ivec, remv + fulld[d]])
                    plsc.store_scatter(out, [ivec, fulld[d]], col)
                return carry

            lax.fori_loop(0, _RC // _LANES, group, 0)

        def pair(p, carry):
            gcp = [None, None]
            wcp = [None, None]
            for h in range(2):
                c = p * 2 + h
                gcp[h] = pltpu.async_copy(
                    tab_hbm.at[slab_v.at[pl.ds(c * _RC, _RC)]],
                    raws[h], gsem[h])
            for h in range(2):
                c = p * 2 + h
                gcp[h].wait()
                select(c, raws[h], outs[h])
                wcp[h] = pltpu.async_copy(
                    outs[h], x_hbm.at[pl.ds(base + c * _RC, _RC)], wsem[h])
            for h in range(2):
                wcp[h].wait()
            return carry

        lax.fori_loop(0, npair, pair, 0)

    return gather_kernel(idx_flat, offs, tab2)


def _tc_matmul(x2, m2, wr, bb=2048):
    B, K = x2.shape
    A = wr.shape[-1]

    def body(x_ref, m_ref, w_ref, o_ref):
        wm = w_ref[...] * m_ref[...]
        o_ref[...] = jnp.dot(x_ref[...], wm,
                             preferred_element_type=jnp.float32)

    return pl.pallas_call(
        body,
        grid=(B // bb,),
        in_specs=[
            pl.BlockSpec((bb, K), lambda n: (n, 0)),
            pl.BlockSpec((K, 1), lambda n: (0, 0)),
            pl.BlockSpec((K, A), lambda n: (0, 0)),
        ],
        out_specs=pl.BlockSpec((bb, A), lambda n: (n, 0)),
        out_shape=jax.ShapeDtypeStruct((B, A), jnp.float32),
    )(x2, m2, wr)


def kernel(inputs, tables, mask, weight):
    B, F = inputs.shape
    _, V, D = tables.shape
    A = weight.shape[-1]
    info = plsc.get_sparse_core_info()
    nw = info.num_cores * info.num_subcores

    idx_flat = jnp.asarray(inputs, jnp.int32).reshape(B * F)
    offs = jnp.tile(jnp.arange(F, dtype=jnp.int32) * V, (B * F // nw) // F)
    tab2 = tables.reshape(F * V // 8, 8 * D)

    x = _sc_gather(idx_flat, offs, tab2, D,
                   info.num_cores, info.num_subcores)
    x2 = x.reshape(B, F * D)
    return _tc_matmul(x2, mask.reshape(F * D, 1), weight.reshape(F * D, A))


# speedup vs baseline: 2.5995x; 2.5995x over previous
import functools

import jax
import jax.numpy as jnp
from jax import lax
from jax.experimental import pallas as pl
from jax.experimental.pallas import tpu as pltpu
from jax.experimental.pallas import tpu_sc as plsc

_LANES = 16
_CH = 128


def _sc_gather(idx_flat, offs, tab1, V, D, nc, ns):
    BF = idx_flat.shape[0]
    nw = nc * ns
    epw = BF // nw
    nvec = epw // _LANES
    npair = epw // (2 * _CH)
    mesh = plsc.VectorSubcoreMesh(core_axis_name="c", subcore_axis_name="s")

    @functools.partial(
        pl.kernel,
        mesh=mesh,
        out_type=jax.ShapeDtypeStruct((BF * D,), jnp.float32),
        scratch_types=[
            pltpu.VMEM((epw,), jnp.int32),
            pltpu.VMEM((epw,), jnp.int32),
            pltpu.VMEM((_CH * D,), jnp.int32),
            pltpu.VMEM((_CH * D,), jnp.int32),
            pltpu.VMEM((_CH * D,), jnp.float32),
            pltpu.VMEM((_CH * D,), jnp.float32),
            pltpu.SemaphoreType.DMA,
            pltpu.SemaphoreType.DMA,
            pltpu.SemaphoreType.DMA,
            pltpu.SemaphoreType.DMA,
        ],
        compiler_params=pltpu.CompilerParams(
            use_tc_tiling_on_sc=False, needs_layout_passes=False),
    )
    def gather_kernel(idx_hbm, offs_hbm, tab_hbm, x_hbm,
                      base_v, offs_v, e0, e1, raw0, raw1, g0, g1, w0, w1):
        wid = lax.axis_index("s") * nc + lax.axis_index("c")
        base = wid * epw
        pltpu.sync_copy(idx_hbm.at[pl.ds(base, epw)], base_v)
        pltpu.sync_copy(offs_hbm, offs_v)

        lane = lax.iota(jnp.int32, _LANES)

        def prep(k, carry):
            o = k * _LANES
            base_v[pl.ds(o, _LANES)] = (
                base_v[pl.ds(o, _LANES)] + offs_v[pl.ds(o, _LANES)])
            return carry

        lax.fori_loop(0, nvec, prep, 0)

        eids = (e0, e1)
        raws = (raw0, raw1)
        gsem = (g0, g1)
        wsem = (w0, w1)

        def expand(c, eid):
            def group(m, carry):
                gvec = base_v[pl.ds(c * _CH + m * _LANES, _LANES)]
                pos = (m * _LANES + lane) * D
                for d in range(D):
                    plsc.store_scatter(eid, [pos + d], gvec + d * V)
                return carry

            lax.fori_loop(0, _CH // _LANES, group, 0)

        def pair(p, carry):
            gcp = [None, None]
            wcp = [None, None]
            for h in range(2):
                c = p * 2 + h
                expand(c, eids[h])
                gcp[h] = pltpu.async_copy(
                    tab_hbm.at[eids[h]], raws[h], gsem[h])
            for h in range(2):
                c = p * 2 + h
                gcp[h].wait()
                wcp[h] = pltpu.async_copy(
                    raws[h],
                    x_hbm.at[pl.ds((base + c * _CH) * D, _CH * D)],
                    wsem[h])
            for h in range(2):
                wcp[h].wait()
            return carry

        lax.fori_loop(0, npair, pair, 0)

    return gather_kernel(idx_flat, offs, tab1)


def _tc_matmul(x2, m2, wr, bb=2048):
    B, K = x2.shape
    A = wr.shape[-1]

    def body(x_ref, m_ref, w_ref, o_ref):
        wm = w_ref[...] * m_ref[...]
        o_ref[...] = jnp.dot(x_ref[...], wm,
                             preferred_element_type=jnp.float32)

    return pl.pallas_call(
        body,
        grid=(B // bb,),
        in_specs=[
            pl.BlockSpec((bb, K), lambda n: (n, 0)),
            pl.BlockSpec((K, 1), lambda n: (0, 0)),
            pl.BlockSpec((K, A), lambda n: (0, 0)),
        ],
        out_specs=pl.BlockSpec((bb, A), lambda n: (n, 0)),
        out_shape=jax.ShapeDtypeStruct((B, A), jnp.float32),
    )(x2, m2, wr)


def kernel(inputs, tables, mask, weight):
    B, F = inputs.shape
    _, V, D = tables.shape
    A = weight.shape[-1]
    info = plsc.get_sparse_core_info()
    nw = info.num_cores * info.num_subcores

    idx_flat = jnp.asarray(inputs, jnp.int32).reshape(B * F)
    offs = jnp.tile(jnp.arange(F, dtype=jnp.int32) * (D * V),
                    (B * F // nw) // F)
    tab1 = jnp.swapaxes(tables, 1, 2).reshape(F * D * V)

    x = _sc_gather(idx_flat, offs, tab1, V, D,
                   info.num_cores, info.num_subcores)
    x2 = x.reshape(B, F * D)
    return _tc_matmul(x2, mask.reshape(F * D, 1), weight.reshape(F * D, A))


# speedup vs baseline: 2.6446x; 1.0173x over previous
import functools

import jax
import jax.numpy as jnp
from jax import lax
from jax.experimental import pallas as pl
from jax.experimental.pallas import tpu as pltpu
from jax.experimental.pallas import tpu_sc as plsc

_LANES = 16
_CH = 512


def _sc_gather(idx_flat, offs, tab1, V, D, nc, ns):
    BF = idx_flat.shape[0]
    nw = nc * ns
    epw = BF // nw
    nvec = epw // _LANES
    npair = epw // (2 * _CH)
    mesh = plsc.VectorSubcoreMesh(core_axis_name="c", subcore_axis_name="s")

    @functools.partial(
        pl.kernel,
        mesh=mesh,
        out_type=jax.ShapeDtypeStruct((BF * D,), jnp.float32),
        scratch_types=[
            pltpu.VMEM((epw,), jnp.int32),
            pltpu.VMEM((epw,), jnp.int32),
            pltpu.VMEM((_CH * D,), jnp.int32),
            pltpu.VMEM((_CH * D,), jnp.int32),
            pltpu.VMEM((_CH * D,), jnp.float32),
            pltpu.VMEM((_CH * D,), jnp.float32),
            pltpu.SemaphoreType.DMA,
            pltpu.SemaphoreType.DMA,
            pltpu.SemaphoreType.DMA,
            pltpu.SemaphoreType.DMA,
        ],
        compiler_params=pltpu.CompilerParams(
            use_tc_tiling_on_sc=False, needs_layout_passes=False),
    )
    def gather_kernel(idx_hbm, offs_hbm, tab_hbm, x_hbm,
                      base_v, offs_v, e0, e1, raw0, raw1, g0, g1, w0, w1):
        wid = lax.axis_index("s") * nc + lax.axis_index("c")
        base = wid * epw
        pltpu.sync_copy(idx_hbm.at[pl.ds(base, epw)], base_v)
        pltpu.sync_copy(offs_hbm, offs_v)

        lane = lax.iota(jnp.int32, _LANES)

        def prep(k, carry):
            o = k * _LANES
            base_v[pl.ds(o, _LANES)] = (
                base_v[pl.ds(o, _LANES)] + offs_v[pl.ds(o, _LANES)])
            return carry

        lax.fori_loop(0, nvec, prep, 0)

        eids = (e0, e1)
        raws = (raw0, raw1)
        gsem = (g0, g1)
        wsem = (w0, w1)

        def expand(c, eid):
            def group(m, carry):
                gvec = base_v[pl.ds(c * _CH + m * _LANES, _LANES)]
                pos = (m * _LANES + lane) * D
                for d in range(D):
                    plsc.store_scatter(eid, [pos + d], gvec + d * V)
                return carry

            lax.fori_loop(0, _CH // _LANES, group, 0)

        def pair(p, carry):
            gcp = [None, None]
            wcp = [None, None]
            for h in range(2):
                c = p * 2 + h
                expand(c, eids[h])
                gcp[h] = pltpu.async_copy(
                    tab_hbm.at[eids[h]], raws[h], gsem[h])
            for h in range(2):
                c = p * 2 + h
                gcp[h].wait()
                wcp[h] = pltpu.async_copy(
                    raws[h],
                    x_hbm.at[pl.ds((base + c * _CH) * D, _CH * D)],
                    wsem[h])
            for h in range(2):
                wcp[h].wait()
            return carry

        lax.fori_loop(0, npair, pair, 0)

    return gather_kernel(idx_flat, offs, tab1)


def _tc_matmul(x2, m2, wr, bb=2048):
    B, K = x2.shape
    A = wr.shape[-1]

    def body(x_ref, m_ref, w_ref, o_ref):
        wm = w_ref[...] * m_ref[...]
        o_ref[...] = jnp.dot(x_ref[...], wm,
                             preferred_element_type=jnp.float32)

    return pl.pallas_call(
        body,
        grid=(B // bb,),
        in_specs=[
            pl.BlockSpec((bb, K), lambda n: (n, 0)),
            pl.BlockSpec((K, 1), lambda n: (0, 0)),
            pl.BlockSpec((K, A), lambda n: (0, 0)),
        ],
        out_specs=pl.BlockSpec((bb, A), lambda n: (n, 0)),
        out_shape=jax.ShapeDtypeStruct((B, A), jnp.float32),
    )(x2, m2, wr)


def kernel(inputs, tables, mask, weight):
    B, F = inputs.shape
    _, V, D = tables.shape
    A = weight.shape[-1]
    info = plsc.get_sparse_core_info()
    nw = info.num_cores * info.num_subcores

    idx_flat = jnp.asarray(inputs, jnp.int32).reshape(B * F)
    offs = jnp.tile(jnp.arange(F, dtype=jnp.int32) * (D * V),
                    (B * F // nw) // F)
    tab1 = jnp.swapaxes(tables, 1, 2).reshape(F * D * V)

    x = _sc_gather(idx_flat, offs, tab1, V, D,
                   info.num_cores, info.num_subcores)
    x2 = x.reshape(B, F * D)
    return _tc_matmul(x2, mask.reshape(F * D, 1), weight.reshape(F * D, A))
